# baseline (device time: 21300 ns/iter reference)
import jax
import jax.numpy as jnp
from jax import lax
from jax.experimental import pallas as pl
from jax.experimental.pallas import tpu as pltpu

N_DEV = 16
NP = N_DEV - 1


def kernel(x):
    m, n = x.shape
    rows = m // N_DEV
    hrows = rows // 2

    def body(x_ref, out_ref, work, comm_ref, send_sems, recv_sems):
        my = lax.axis_index("i")

        barrier_sem = pltpu.get_barrier_semaphore()
        for o in range(1, N_DEV):
            pl.semaphore_signal(
                barrier_sem, inc=1,
                device_id=(lax.rem(my + o, N_DEV),),
                device_id_type=pl.DeviceIdType.MESH,
            )
        pl.semaphore_wait(barrier_sem, NP)

        work[...] = x_ref[...].astype(jnp.bfloat16)

        def half(ref, p, hb):
            return ref.at[pl.ds(p * rows + hb * hrows, hrows), :]

        def rs_rdma(o, hb):
            r = lax.rem(my - o + N_DEV, N_DEV)
            s = hb * NP + (o - 1)
            return pltpu.make_async_remote_copy(
                src_ref=half(work, r, hb),
                dst_ref=comm_ref.at[s],
                send_sem=send_sems.at[s],
                recv_sem=recv_sems.at[s],
                device_id=(r,),
                device_id_type=pl.DeviceIdType.MESH,
            )

        def ag_rdma(o, hb):
            r = lax.rem(my + o, N_DEV)
            s = (2 + hb) * NP + (o - 1)
            return pltpu.make_async_remote_copy(
                src_ref=half(work, my, hb),
                dst_ref=half(work, my, hb),
                send_sem=send_sems.at[s],
                recv_sem=recv_sems.at[s],
                device_id=(r,),
                device_id_type=pl.DeviceIdType.MESH,
            )

        rs = {0: [], 1: []}
        for hb in (0, 1):
            for o in range(1, N_DEV):
                rdma = rs_rdma(o, hb)
                rdma.start()
                rs[hb].append(rdma)

        ag = {0: [], 1: []}
        for hb in (0, 1):
            for rdma in rs[hb]:
                rdma.wait()
            idx = my * rows + hb * hrows
            acc = work[pl.ds(idx, hrows), :]
            for o in range(1, N_DEV):
                acc = acc + comm_ref[hb * NP + (o - 1)]
            work[pl.ds(idx, hrows), :] = acc
            for o in range(1, N_DEV):
                rdma = ag_rdma(o, hb)
                rdma.start()
                ag[hb].append(rdma)

        for hb in (0, 1):
            for rdma in ag[hb]:
                rdma.wait()

        out_ref[...] = work[...].astype(x_ref.dtype)

    n_sems = 4 * NP
    return pl.pallas_call(
        body,
        out_shape=jax.ShapeDtypeStruct((m, n), x.dtype),
        in_specs=[pl.BlockSpec(memory_space=pltpu.VMEM)],
        out_specs=pl.BlockSpec(memory_space=pltpu.VMEM),
        scratch_shapes=[
            pltpu.VMEM((m, n), jnp.bfloat16),
            pltpu.VMEM((2 * NP, hrows, n), jnp.bfloat16),
            pltpu.SemaphoreType.DMA((n_sems,)),
            pltpu.SemaphoreType.DMA((n_sems,)),
        ],
        compiler_params=pltpu.CompilerParams(collective_id=0),
    )(x)


# device time: 20408 ns/iter; 1.0437x vs baseline; 1.0437x over previous
import jax
import jax.numpy as jnp
from jax import lax
from jax.experimental import pallas as pl
from jax.experimental.pallas import tpu as pltpu

N_DEV = 16


def kernel(x):
    m, n = x.shape
    rows = m // N_DEV

    def body(x_ref, out_ref, work, comm_ref, send_sems, recv_sems):
        my = lax.axis_index("i")

        barrier_sem = pltpu.get_barrier_semaphore()
        for o in range(1, N_DEV):
            pl.semaphore_signal(
                barrier_sem, inc=1,
                device_id=(lax.rem(my + o, N_DEV),),
                device_id_type=pl.DeviceIdType.MESH,
            )
        pl.semaphore_wait(barrier_sem, N_DEV - 1)

        work[...] = x_ref[...].astype(jnp.bfloat16)

        def peer_chunk(ref, p):
            return ref.at[pl.ds(p * rows, rows), :]

        rs = []
        for o in range(1, N_DEV):
            r = lax.rem(my - o + N_DEV, N_DEV)
            rdma = pltpu.make_async_remote_copy(
                src_ref=peer_chunk(work, r),
                dst_ref=comm_ref.at[o - 1],
                send_sem=send_sems.at[o - 1],
                recv_sem=recv_sems.at[o - 1],
                device_id=(r,),
                device_id_type=pl.DeviceIdType.MESH,
            )
            rdma.start()
            rs.append(rdma)
        for rdma in rs:
            rdma.wait()
        acc = peer_chunk(work, my)[...]
        for o in range(1, N_DEV):
            acc = acc + comm_ref[o - 1]
        work[pl.ds(my * rows, rows), :] = acc

        ag = []
        for o in range(1, N_DEV):
            r = lax.rem(my + o, N_DEV)
            rdma = pltpu.make_async_remote_copy(
                src_ref=peer_chunk(work, my),
                dst_ref=peer_chunk(work, my),
                send_sem=send_sems.at[N_DEV - 1 + o - 1],
                recv_sem=recv_sems.at[N_DEV - 1 + o - 1],
                device_id=(r,),
                device_id_type=pl.DeviceIdType.MESH,
            )
            rdma.start()
            ag.append(rdma)
        for rdma in ag:
            rdma.wait()

        out_ref[...] = work[...].astype(x_ref.dtype)

    n_sems = 2 * (N_DEV - 1)
    return pl.pallas_call(
        body,
        out_shape=jax.ShapeDtypeStruct((m, n), x.dtype),
        in_specs=[pl.BlockSpec(memory_space=pltpu.VMEM)],
        out_specs=pl.BlockSpec(memory_space=pltpu.VMEM),
        scratch_shapes=[
            pltpu.VMEM((m, n), jnp.bfloat16),
            pltpu.VMEM((N_DEV - 1, rows, n), jnp.bfloat16),
            pltpu.SemaphoreType.DMA((n_sems,)),
            pltpu.SemaphoreType.DMA((n_sems,)),
        ],
        compiler_params=pltpu.CompilerParams(collective_id=0),
    )(x)
